# baseline (speedup 1.0000x reference)
import jax
import jax.numpy as jnp
from jax import lax
from jax.experimental import pallas as pl
from jax.experimental.pallas import tpu as pltpu
from jax.experimental.pallas import tpu_sc as plsc

N_CLS = 10000
N_CTX = 16
D = 512
SEQ = 77
SUF = 60
B = 1024

NC = 2
NS = 16
NW = NC * NS
BPW = B // NW
NPAIR = BPW // 2
NCH = D // 16


def _row_copy(dst_ref, dst_row, src_ref, src_row):
    for c in range(NCH):
        dst_ref[0, dst_row, pl.ds(c * 16, 16)] = src_ref[0, src_row, pl.ds(c * 16, 16)]


def _sc_body(idx_hbm, ctx_hbm, pre_hbm, suf_hbm, out_hbm,
             idx_v, ctx_v, pre_v0, pre_v1, suf_v0, suf_v1, combo_v,
             gsem0, gsem1):
    wid = lax.axis_index("s") * NC + lax.axis_index("c")
    base = wid * BPW
    pltpu.sync_copy(idx_hbm.at[pl.ds(base, BPW)], idx_v)
    pltpu.sync_copy(ctx_hbm, ctx_v)

    def place_ctx(r, carry):
        for c in range(NCH):
            combo_v[0, 1 + r, pl.ds(c * 16, 16)] = ctx_v[r, pl.ds(c * 16, 16)]
        return carry

    lax.fori_loop(0, N_CTX, place_ctx, 0)

    vec0 = idx_v[pl.ds(0, 16)]
    vec1 = idx_v[pl.ds(16, 16)]
    lanes = lax.iota(jnp.int32, 16)

    def extract(i):
        sel = jnp.where(i < 16, vec0, vec1)
        return jnp.sum(jnp.where(lanes == (i % 16), sel, 0))

    pre_bufs = (pre_v0, pre_v1)
    suf_bufs = (suf_v0, suf_v1)
    gsems = (gsem0, gsem1)

    def issue(i, j):
        s = extract(i)
        pltpu.async_copy(pre_hbm.at[pl.ds(s, 1)], pre_bufs[j], gsems[j])
        pltpu.async_copy(suf_hbm.at[pl.ds(s, 1)], suf_bufs[j], gsems[j])

    issue(0, 0)
    issue(1, 1)

    def body(p, carry):
        for j in range(2):
            i = 2 * p + j
            pltpu.make_async_copy(
                pre_hbm.at[pl.ds(0, 1)], pre_bufs[j], gsems[j]).wait()
            pltpu.make_async_copy(
                suf_hbm.at[pl.ds(0, 1)], suf_bufs[j], gsems[j]).wait()
            _row_copy(combo_v, 0, pre_bufs[j], 0)

            def place_suf(r, c2, j=j):
                _row_copy(combo_v, 1 + N_CTX + r, suf_bufs[j], r)
                return c2

            lax.fori_loop(0, SUF, place_suf, 0)

            @pl.when(p < NPAIR - 1)
            def _():
                issue(i + 2, j)

            pltpu.sync_copy(combo_v, out_hbm.at[pl.ds(base + i, 1)])
        return carry

    lax.fori_loop(0, NPAIR, body, 0)


@jax.jit
def _launch(idx, ctx, token_prefix, token_suffix):
    call = pl.kernel(
        _sc_body,
        out_type=jax.ShapeDtypeStruct((B, SEQ, D), jnp.float32),
        mesh=plsc.VectorSubcoreMesh(core_axis_name="c", subcore_axis_name="s"),
        compiler_params=pltpu.CompilerParams(needs_layout_passes=False),
        scratch_types=[
            pltpu.VMEM((BPW,), jnp.int32),
            pltpu.VMEM((N_CTX, D), jnp.float32),
            pltpu.VMEM((1, 1, D), jnp.float32),
            pltpu.VMEM((1, 1, D), jnp.float32),
            pltpu.VMEM((1, SUF, D), jnp.float32),
            pltpu.VMEM((1, SUF, D), jnp.float32),
            pltpu.VMEM((1, SEQ, D), jnp.float32),
            pltpu.SemaphoreType.DMA,
            pltpu.SemaphoreType.DMA,
        ],
    )
    return call(idx, ctx, token_prefix, token_suffix)


def kernel(idx, ctx, token_prefix, token_suffix):
    return _launch(idx, ctx, token_prefix, token_suffix)


# speedup vs baseline: 1.1282x; 1.1282x over previous
import jax
import jax.numpy as jnp
from jax import lax
from jax.experimental import pallas as pl
from jax.experimental.pallas import tpu as pltpu
from jax.experimental.pallas import tpu_sc as plsc

N_CLS = 10000
N_CTX = 16
D = 512
SEQ = 77
SUF = 60
B = 1024

NC = 2
NS = 16
NW = NC * NS
BPW = B // NW
NPAIR = BPW // 2
NCH = D // 16


def _row_copy(dst_ref, dst_row, src_ref, src_row):
    for c in range(NCH):
        dst_ref[0, dst_row, pl.ds(c * 16, 16)] = src_ref[0, src_row, pl.ds(c * 16, 16)]


def _sc_body(idx_hbm, ctx_hbm, pre_hbm, suf_hbm, out_hbm,
             idx_v, ctx_v, pre_v0, pre_v1, suf_v0, suf_v1, combo_v,
             gsem0, gsem1):
    wid = lax.axis_index("s") * NC + lax.axis_index("c")
    base = wid * BPW
    pltpu.sync_copy(idx_hbm.at[pl.ds(base, BPW)], idx_v)
    pltpu.sync_copy(ctx_hbm, ctx_v)

    def place_ctx(r, carry):
        for c in range(NCH):
            combo_v[0, 1 + r, pl.ds(c * 16, 16)] = ctx_v[r, pl.ds(c * 16, 16)]
        return carry

    lax.fori_loop(0, N_CTX, place_ctx, 0)

    vec0 = idx_v[pl.ds(0, 16)]
    vec1 = idx_v[pl.ds(16, 16)]
    lanes = lax.iota(jnp.int32, 16)

    def extract(i):
        sel = jnp.where(i < 16, vec0, vec1)
        return jnp.sum(jnp.where(lanes == (i % 16), sel, 0))

    pre_bufs = (pre_v0, pre_v1)
    suf_bufs = (suf_v0, suf_v1)
    gsems = (gsem0, gsem1)

    def issue(i, j):
        s = extract(i)
        pltpu.async_copy(pre_hbm.at[pl.ds(s, 1)], pre_bufs[j], gsems[j])
        pltpu.async_copy(suf_hbm.at[pl.ds(s, 1)], suf_bufs[j], gsems[j])

    issue(0, 0)
    issue(1, 1)

    def body(p, carry):
        for j in range(2):
            i = 2 * p + j
            pltpu.make_async_copy(
                pre_hbm.at[pl.ds(0, 1)], pre_bufs[j], gsems[j]).wait()
            pltpu.make_async_copy(
                suf_hbm.at[pl.ds(0, 1)], suf_bufs[j], gsems[j]).wait()
            _row_copy(combo_v, 0, pre_bufs[j], 0)

            @plsc.parallel_loop(0, SUF, unroll=4)
            def place_suf(r, j=j):
                _row_copy(combo_v, 1 + N_CTX + r, suf_bufs[j], r)

            @pl.when(p < NPAIR - 1)
            def _():
                issue(i + 2, j)

            pltpu.sync_copy(combo_v, out_hbm.at[pl.ds(base + i, 1)])
        return carry

    lax.fori_loop(0, NPAIR, body, 0)


@jax.jit
def _launch(idx, ctx, token_prefix, token_suffix):
    call = pl.kernel(
        _sc_body,
        out_type=jax.ShapeDtypeStruct((B, SEQ, D), jnp.float32),
        mesh=plsc.VectorSubcoreMesh(core_axis_name="c", subcore_axis_name="s"),
        compiler_params=pltpu.CompilerParams(needs_layout_passes=False),
        scratch_types=[
            pltpu.VMEM((BPW,), jnp.int32),
            pltpu.VMEM((N_CTX, D), jnp.float32),
            pltpu.VMEM((1, 1, D), jnp.float32),
            pltpu.VMEM((1, 1, D), jnp.float32),
            pltpu.VMEM((1, SUF, D), jnp.float32),
            pltpu.VMEM((1, SUF, D), jnp.float32),
            pltpu.VMEM((1, SEQ, D), jnp.float32),
            pltpu.SemaphoreType.DMA,
            pltpu.SemaphoreType.DMA,
        ],
    )
    return call(idx, ctx, token_prefix, token_suffix)


def kernel(idx, ctx, token_prefix, token_suffix):
    return _launch(idx, ctx, token_prefix, token_suffix)
